# baseline (speedup 1.0000x reference)
import functools

import jax
import jax.numpy as jnp
from jax.experimental import pallas as pl
import jax.experimental.pallas.tpu as pltpu

_C = 96


def _nonzero_kernel(idx_ref, sel_ref):
    def init(j, carry):
        sel_ref[j] = 0
        return carry

    jax.lax.fori_loop(0, _C, init, 0)

    def body(i, count):
        nz = idx_ref[i] != 0.0

        @pl.when(nz)
        def _():
            sel_ref[count] = i

        return count + nz.astype(jnp.int32)

    jax.lax.fori_loop(0, _C, body, 0)


def _gather_kernel(sel_ref, x_ref, o_ref):
    o_ref[...] = x_ref[...]


@jax.jit
def kernel(input_tensor, indexes):
    b, c, h, w = input_tensor.shape
    sel = pl.pallas_call(
        _nonzero_kernel,
        in_specs=[pl.BlockSpec(memory_space=pltpu.SMEM)],
        out_specs=pl.BlockSpec(memory_space=pltpu.SMEM),
        out_shape=jax.ShapeDtypeStruct((c,), jnp.int32),
    )(indexes)

    hw = h * w
    lanes = 128
    sub = hw // lanes
    x = input_tensor.reshape(b, c, sub, lanes)

    grid_spec = pltpu.PrefetchScalarGridSpec(
        num_scalar_prefetch=1,
        grid=(c,),
        in_specs=[
            pl.BlockSpec((b, 1, sub, lanes), lambda j, sel: (0, sel[j], 0, 0)),
        ],
        out_specs=pl.BlockSpec((b, 1, sub, lanes), lambda j, sel: (0, j, 0, 0)),
    )
    out = pl.pallas_call(
        _gather_kernel,
        grid_spec=grid_spec,
        out_shape=jax.ShapeDtypeStruct((b, c, sub, lanes), jnp.float32),
    )(sel, x)
    return out.reshape(b, c, h, w)
